# baseline (speedup 1.0000x reference)
import functools

import jax
import jax.numpy as jnp
from jax import lax
from jax.experimental import pallas as pl
from jax.experimental.pallas import tpu as pltpu
from jax.experimental.pallas import tpu_sc as plsc

D = 512
TN = 2048


def _sc_gather(wte, ids):
    info = plsc.get_sparse_core_info()
    nw = info.num_cores * info.num_subcores
    b = ids.shape[0]
    b_per_w = b // nw
    mesh = plsc.VectorSubcoreMesh(core_axis_name="c", subcore_axis_name="s")

    @functools.partial(
        pl.kernel,
        mesh=mesh,
        out_type=jax.ShapeDtypeStruct((b, D), jnp.float32),
        scratch_types=[
            pltpu.VMEM((b_per_w,), jnp.int32),
            pltpu.VMEM((b_per_w, D), jnp.float32),
            pltpu.SemaphoreType.DMA,
        ],
    )
    def gather_kernel(idx_hbm, table_hbm, out_hbm, idx_v, rows_v, sem):
        wid = lax.axis_index("s") * info.num_cores + lax.axis_index("c")
        base = wid * b_per_w
        pltpu.sync_copy(idx_hbm.at[pl.ds(base, b_per_w)], idx_v)
        pltpu.async_copy(table_hbm.at[idx_v], rows_v, sem).wait()
        pltpu.sync_copy(rows_v, out_hbm.at[pl.ds(base, b_per_w)])

    return gather_kernel(ids, wte)


def _mlp_body(x_ref, w1_ref, b1_ref, w2_ref, b2_ref, ht_ref):
    x = x_ref[...].astype(jnp.bfloat16)
    a = jnp.dot(x, w1_ref[...].astype(jnp.bfloat16),
                preferred_element_type=jnp.float32) + b1_ref[...]
    a = jnp.maximum(a, 0.0).astype(jnp.bfloat16)
    ht = lax.dot_general(w2_ref[...].astype(jnp.bfloat16), a,
                         (((0,), (1,)), ((), ())),
                         preferred_element_type=jnp.float32)
    ht_ref[...] = (ht + b2_ref[...]).astype(jnp.bfloat16)


def _mlp_t(x, w1, b1, w2, b2):
    m = x.shape[0]
    return pl.pallas_call(
        _mlp_body,
        out_shape=jax.ShapeDtypeStruct((D, m), jnp.bfloat16),
    )(x, w1, b1.reshape(1, D), w2, b2.reshape(D, 1))


def _head_body(ht_ref, wht_ref, bh_ref, out_ref):
    bh_col = jnp.transpose(bh_ref[...])
    out_ref[...] = jnp.dot(wht_ref[...].astype(jnp.bfloat16), ht_ref[...],
                           preferred_element_type=jnp.float32) + bh_col


def _head_t(ht, wht, bh_row):
    m = ht.shape[1]
    v = wht.shape[0]
    grid = (pl.cdiv(v, TN),)
    return pl.pallas_call(
        _head_body,
        grid=grid,
        in_specs=[
            pl.BlockSpec((D, m), lambda i: (0, 0)),
            pl.BlockSpec((TN, D), lambda i: (i, 0)),
            pl.BlockSpec((1, TN), lambda i: (0, i)),
        ],
        out_specs=pl.BlockSpec((TN, m), lambda i: (i, 0)),
        out_shape=jax.ShapeDtypeStruct((v, m), jnp.float32),
        compiler_params=pltpu.CompilerParams(vmem_limit_bytes=63 * 1024 * 1024),
    )(ht, wht, bh_row)


def kernel(input_ids, wte, W1, b1, W2, b2, Wh, bh):
    ids = input_ids.reshape(-1).astype(jnp.int32)
    v = Wh.shape[1]
    x = _sc_gather(wte, ids)
    ht = _mlp_t(x, W1, b1, W2, b2)
    logits_t = _head_t(ht, Wh.T, bh.reshape(1, v))
    return logits_t.T.reshape(input_ids.shape + (v,))


# speedup vs baseline: 1.0149x; 1.0149x over previous
import functools

import jax
import jax.numpy as jnp
from jax import lax
from jax.experimental import pallas as pl
from jax.experimental.pallas import tpu as pltpu
from jax.experimental.pallas import tpu_sc as plsc

D = 512
TN = 2048


def _sc_gather(wte, ids):
    info = plsc.get_sparse_core_info()
    nw = info.num_cores * info.num_subcores
    b = ids.shape[0]
    b_per_w = b // nw
    mesh = plsc.VectorSubcoreMesh(core_axis_name="c", subcore_axis_name="s")

    @functools.partial(
        pl.kernel,
        mesh=mesh,
        out_type=jax.ShapeDtypeStruct((b, D), jnp.float32),
        scratch_types=[
            pltpu.VMEM((b_per_w,), jnp.int32),
            pltpu.VMEM((b_per_w, D), jnp.float32),
            pltpu.SemaphoreType.DMA,
        ],
    )
    def gather_kernel(idx_hbm, table_hbm, out_hbm, idx_v, rows_v, sem):
        wid = lax.axis_index("s") * info.num_cores + lax.axis_index("c")
        base = wid * b_per_w
        pltpu.sync_copy(idx_hbm.at[pl.ds(base, b_per_w)], idx_v)
        pltpu.async_copy(table_hbm.at[idx_v], rows_v, sem).wait()
        pltpu.sync_copy(rows_v, out_hbm.at[pl.ds(base, b_per_w)])

    return gather_kernel(ids, wte)


def _mlp_head_body(x_ref, w1_ref, b1_ref, w2_ref, b2_ref, wht_ref, bh_ref,
                   out_ref, ht_ref):
    @pl.when(pl.program_id(0) == 0)
    def _():
        x = x_ref[...].astype(jnp.bfloat16)
        a = jnp.dot(x, w1_ref[...].astype(jnp.bfloat16),
                    preferred_element_type=jnp.float32) + b1_ref[...]
        a = jnp.maximum(a, 0.0).astype(jnp.bfloat16)
        ht = lax.dot_general(w2_ref[...].astype(jnp.bfloat16), a,
                             (((0,), (1,)), ((), ())),
                             preferred_element_type=jnp.float32)
        ht_ref[...] = (ht + b2_ref[...]).astype(jnp.bfloat16)

    bh_col = jnp.transpose(bh_ref[...])
    out_ref[...] = jnp.dot(wht_ref[...].astype(jnp.bfloat16), ht_ref[...],
                           preferred_element_type=jnp.float32) + bh_col


def _mlp_head_t(x, w1, b1, w2, b2, wht, bh_row):
    m = x.shape[0]
    v = wht.shape[0]
    grid = (pl.cdiv(v, TN),)
    return pl.pallas_call(
        _mlp_head_body,
        grid=grid,
        in_specs=[
            pl.BlockSpec((m, D), lambda i: (0, 0)),
            pl.BlockSpec((D, D), lambda i: (0, 0)),
            pl.BlockSpec((1, D), lambda i: (0, 0)),
            pl.BlockSpec((D, D), lambda i: (0, 0)),
            pl.BlockSpec((D, 1), lambda i: (0, 0)),
            pl.BlockSpec((TN, D), lambda i: (i, 0)),
            pl.BlockSpec((1, TN), lambda i: (0, i)),
        ],
        out_specs=pl.BlockSpec((TN, m), lambda i: (i, 0)),
        out_shape=jax.ShapeDtypeStruct((v, m), jnp.float32),
        scratch_shapes=[pltpu.VMEM((D, m), jnp.bfloat16)],
        compiler_params=pltpu.CompilerParams(vmem_limit_bytes=63 * 1024 * 1024),
    )(x, w1, b1.reshape(1, D), w2, b2.reshape(D, 1), wht, bh_row)


def kernel(input_ids, wte, W1, b1, W2, b2, Wh, bh):
    ids = input_ids.reshape(-1).astype(jnp.int32)
    v = Wh.shape[1]
    x = _sc_gather(wte, ids)
    logits_t = _mlp_head_t(x, W1, b1, W2, b2, Wh.T, bh.reshape(1, v))
    return logits_t.T.reshape(input_ids.shape + (v,))


# speedup vs baseline: 1.0159x; 1.0010x over previous
import functools

import jax
import jax.numpy as jnp
from jax import lax
from jax.experimental import pallas as pl
from jax.experimental.pallas import tpu as pltpu
from jax.experimental.pallas import tpu_sc as plsc

D = 512
TN = 2048


def _sc_gather(wte, ids):
    info = plsc.get_sparse_core_info()
    nw = info.num_cores * info.num_subcores
    b = ids.shape[0]
    b_per_w = b // nw
    mesh = plsc.VectorSubcoreMesh(core_axis_name="c", subcore_axis_name="s")

    @functools.partial(
        pl.kernel,
        mesh=mesh,
        out_type=jax.ShapeDtypeStruct((b, D), jnp.float32),
        scratch_types=[
            pltpu.VMEM((b_per_w,), jnp.int32),
            pltpu.VMEM((b_per_w, D), jnp.float32),
            pltpu.SemaphoreType.DMA,
        ],
    )
    def gather_kernel(idx_hbm, table_hbm, out_hbm, idx_v, rows_v, sem):
        wid = lax.axis_index("s") * info.num_cores + lax.axis_index("c")
        base = wid * b_per_w
        pltpu.sync_copy(idx_hbm.at[pl.ds(base, b_per_w)], idx_v)
        pltpu.async_copy(table_hbm.at[idx_v], rows_v, sem).wait()
        pltpu.sync_copy(rows_v, out_hbm.at[pl.ds(base, b_per_w)])

    return gather_kernel(ids, wte)


def _mlp_head_body(x_ref, w1_ref, b1_ref, w2_ref, b2_ref, wht_ref, bh_ref,
                   out_ref, ht_ref):
    @pl.when(pl.program_id(0) == 0)
    def _():
        x = x_ref[...].astype(jnp.bfloat16)
        a = jnp.dot(x, w1_ref[...].astype(jnp.bfloat16),
                    preferred_element_type=jnp.float32) + b1_ref[...]
        a = jnp.maximum(a, 0.0).astype(jnp.bfloat16)
        ht = lax.dot_general(w2_ref[...].astype(jnp.bfloat16), a,
                             (((0,), (1,)), ((), ())),
                             preferred_element_type=jnp.float32)
        ht_ref[...] = (ht + jnp.transpose(b2_ref[...])).astype(jnp.bfloat16)

    bh_col = jnp.transpose(bh_ref[...])
    out_ref[...] = jnp.dot(wht_ref[...].astype(jnp.bfloat16), ht_ref[...],
                           preferred_element_type=jnp.float32) + bh_col


def _mlp_head_t(x, w1, b1, w2, b2, wht, bh_row):
    m = x.shape[0]
    v = wht.shape[0]
    grid = (pl.cdiv(v, TN),)
    return pl.pallas_call(
        _mlp_head_body,
        grid=grid,
        in_specs=[
            pl.BlockSpec((m, D), lambda i: (0, 0)),
            pl.BlockSpec((D, D), lambda i: (0, 0)),
            pl.BlockSpec((1, D), lambda i: (0, 0)),
            pl.BlockSpec((D, D), lambda i: (0, 0)),
            pl.BlockSpec((1, D), lambda i: (0, 0)),
            pl.BlockSpec((TN, D), lambda i: (i, 0)),
            pl.BlockSpec((1, TN), lambda i: (0, i)),
        ],
        out_specs=pl.BlockSpec((TN, m), lambda i: (i, 0)),
        out_shape=jax.ShapeDtypeStruct((v, m), jnp.float32),
        scratch_shapes=[pltpu.VMEM((D, m), jnp.bfloat16)],
        compiler_params=pltpu.CompilerParams(vmem_limit_bytes=63 * 1024 * 1024),
    )(x, w1, b1.reshape(1, D), w2, b2.reshape(1, D), wht, bh_row)


def kernel(input_ids, wte, W1, b1, W2, b2, Wh, bh):
    ids = input_ids.reshape(-1).astype(jnp.int32)
    v = Wh.shape[1]
    x = _sc_gather(wte, ids)
    logits_t = _mlp_head_t(x, W1, b1, W2, b2, Wh.T, bh.reshape(1, v))
    return logits_t.T.reshape(input_ids.shape + (v,))
